# Initial kernel scaffold; baseline (speedup 1.0000x reference)
import jax
import jax.numpy as jnp
from jax.experimental import pallas as pl


def kernel(src, test_buffer):
    raise NotImplementedError("write your pallas kernel here")


# speedup vs baseline: 1.0166x; 1.0166x over previous
import jax
import jax.numpy as jnp
from jax.experimental import pallas as pl


def _copy_kernel(src_ref, out_ref):
    out_ref[...] = src_ref[...]


def kernel(src, test_buffer):
    return pl.pallas_call(
        _copy_kernel,
        out_shape=jax.ShapeDtypeStruct(test_buffer.shape, test_buffer.dtype),
    )(src)
